# baseline (speedup 1.0000x reference)
import functools

import jax
import jax.numpy as jnp
from jax import lax
from jax.experimental import pallas as pl
from jax.experimental.pallas import tpu as pltpu
from jax.experimental.pallas import tpu_sc as plsc

EMB = 128
NTYPES = 64
N = 100000
CHUNK = 80
NCHUNKS = N // CHUNK
NW = 32
NB = 3
NFULL = 39
NG = NFULL // NB
CUNROLL = 4
NROWG = CHUNK // 16


def _body(w_hbm, z_hbm, out_hbm, table_v, idx_v, *scratch):
    outs = scratch[0:NB]
    osem = scratch[NB:2 * NB]
    isem = scratch[2 * NB]
    wid = lax.axis_index("s") * 2 + lax.axis_index("c")
    has_extra = wid < NCHUNKS - NFULL * NW

    def chunk_base(t):
        return (wid + t * NW) * CHUNK

    def idx_desc(t):
        return pltpu.make_async_copy(
            z_hbm.at[pl.ds(chunk_base(t), CHUNK)],
            idx_v.at[pl.ds(t * CHUNK, CHUNK)], isem)

    pltpu.make_async_copy(w_hbm, table_v, isem).start()
    for t in range(NFULL):
        idx_desc(t).start()

    @pl.when(has_extra)
    def _():
        idx_desc(NFULL).start()

    pltpu.make_async_copy(w_hbm, table_v, isem).wait()
    for t in range(NFULL):
        idx_desc(t).wait()

    @pl.when(has_extra)
    def _():
        idx_desc(NFULL).wait()

    rvecs = [(lax.iota(jnp.int32, 16) + g * 16) * EMB for g in range(NROWG)]

    def compute_chunk(t, out_b):
        zs = [(idx_v[pl.ds(t * CHUNK + g * 16, 16)] - 1) * EMB
              for g in range(NROWG)]

        def cbody(ci, carry):
            for cc in range(CUNROLL):
                c_vec = jnp.zeros((16,), jnp.int32) + (ci * CUNROLL + cc)
                for g in range(NROWG):
                    v = plsc.load_gather(table_v, [zs[g] + c_vec])
                    plsc.store_scatter(out_b, [rvecs[g] + c_vec], v)
            return carry

        lax.fori_loop(0, EMB // CUNROLL, cbody, 0)

    def out_desc(t, b):
        return pltpu.make_async_copy(
            outs[b], out_hbm.at[pl.ds(chunk_base(t) * EMB, CHUNK * EMB)],
            osem[b])

    for b in range(NB):
        compute_chunk(b, outs[b])
        out_desc(b, b).start()

    def outer(g, carry):
        t0 = g * NB
        for b in range(NB):
            out_desc(t0 - NB + b, b).wait()
            compute_chunk(t0 + b, outs[b])
            out_desc(t0 + b, b).start()
        return carry

    lax.fori_loop(1, NG, outer, 0)

    for b in range(NB):
        out_desc((NG - 1) * NB + b, b).wait()

    @pl.when(has_extra)
    def _():
        compute_chunk(NFULL, outs[0])
        out_desc(NFULL, 0).start()
        out_desc(NFULL, 0).wait()


def kernel(Z, W):
    mesh = plsc.VectorSubcoreMesh(core_axis_name="c", subcore_axis_name="s")
    k = functools.partial(
        pl.kernel,
        mesh=mesh,
        compiler_params=pltpu.CompilerParams(needs_layout_passes=False),
        out_type=jax.ShapeDtypeStruct((N * EMB,), jnp.float32),
        scratch_types=(
            [pltpu.VMEM((NTYPES * EMB,), jnp.float32),
             pltpu.VMEM(((NFULL + 1) * CHUNK,), jnp.int32)]
            + [pltpu.VMEM((CHUNK * EMB,), jnp.float32) for _ in range(NB)]
            + [pltpu.SemaphoreType.DMA for _ in range(NB + 1)]
        ),
    )(_body)
    return k(W.reshape(NTYPES * EMB), Z).reshape(N, EMB)


# speedup vs baseline: 11.3339x; 11.3339x over previous
import functools

import jax
import jax.numpy as jnp
from jax import lax
from jax.experimental import pallas as pl
from jax.experimental.pallas import tpu as pltpu
from jax.experimental.pallas import tpu_sc as plsc

EMB = 128
NROWS = 65
N = 100000
CHUNK = 80
NCHUNKS = N // CHUNK
NW = 32
NB = 3
NFULL = 39
NG = NFULL // NB


def _body(w_hbm, z_hbm, out_hbm, w_sh, idx_v, *scratch):
    rows = scratch[0:NB]
    gsem = scratch[NB:2 * NB]
    osem = scratch[2 * NB:3 * NB]
    isem = scratch[3 * NB]
    wid = lax.axis_index("s") * 2 + lax.axis_index("c")
    has_extra = wid < NCHUNKS - NFULL * NW

    def chunk_base(t):
        return (wid + t * NW) * CHUNK

    def idx_desc(t):
        return pltpu.make_async_copy(
            z_hbm.at[pl.ds(chunk_base(t), CHUNK)], idx_v.at[t], isem)

    @pl.when(lax.axis_index("s") == 0)
    def _():
        pltpu.sync_copy(w_hbm, w_sh)

    for t in range(NFULL):
        idx_desc(t).start()

    @pl.when(has_extra)
    def _():
        idx_desc(NFULL).start()

    for t in range(NFULL):
        idx_desc(t).wait()

    @pl.when(has_extra)
    def _():
        idx_desc(NFULL).wait()

    plsc.subcore_barrier()

    def gather_desc(t, b):
        return pltpu.make_async_copy(w_sh.at[idx_v.at[t]], rows[b], gsem[b])

    def out_desc(t, b):
        return pltpu.make_async_copy(
            rows[b], out_hbm.at[pl.ds(chunk_base(t), CHUNK)], osem[b])

    for b in range(NB):
        gather_desc(b, b).start()

    def outer(g, carry):
        t0 = g * NB
        for b in range(NB):
            gather_desc(t0 + b, b).wait()
            out_desc(t0 + b, b).start()
        @pl.when(g + 1 < NG)
        def _():
            for b in range(NB):
                out_desc(t0 + b, b).wait()
                gather_desc(t0 + NB + b, b).start()
        return carry

    lax.fori_loop(0, NG, outer, 0)

    for b in range(NB):
        out_desc((NG - 1) * NB + b, b).wait()

    @pl.when(has_extra)
    def _():
        gather_desc(NFULL, 0).start()
        gather_desc(NFULL, 0).wait()
        out_desc(NFULL, 0).start()
        out_desc(NFULL, 0).wait()


def kernel(Z, W):
    w_pad = jnp.concatenate([jnp.zeros((1, EMB), W.dtype), W], axis=0)
    mesh = plsc.VectorSubcoreMesh(core_axis_name="c", subcore_axis_name="s")
    k = functools.partial(
        pl.kernel,
        mesh=mesh,
        out_type=jax.ShapeDtypeStruct((N, EMB), jnp.float32),
        scratch_types=(
            [pltpu.VMEM_SHARED((NROWS, EMB), jnp.float32),
             pltpu.VMEM((NFULL + 1, CHUNK), jnp.int32)]
            + [pltpu.VMEM((CHUNK, EMB), jnp.float32) for _ in range(NB)]
            + [pltpu.SemaphoreType.DMA for _ in range(2 * NB + 1)]
        ),
    )(_body)
    return k(w_pad, Z)


# speedup vs baseline: 11.5432x; 1.0185x over previous
import functools

import jax
import jax.numpy as jnp
from jax import lax
from jax.experimental import pallas as pl
from jax.experimental.pallas import tpu as pltpu
from jax.experimental.pallas import tpu_sc as plsc

EMB = 128
NROWS = 65
N = 100000
CHUNK = 80
NCHUNKS = N // CHUNK
NW = 32
GSZ = 3
NSET = 2
NFULL = 39
NG = NFULL // GSZ
NDG = NG // 2


def _body(w_hbm, z_hbm, out_hbm, w_sh, idx_v, *scratch):
    rows = scratch[0:NSET * GSZ]
    gsem = scratch[NSET * GSZ:2 * NSET * GSZ]
    osem = scratch[2 * NSET * GSZ:3 * NSET * GSZ]
    isem = scratch[3 * NSET * GSZ]
    wid = lax.axis_index("s") * 2 + lax.axis_index("c")
    has_extra = wid < NCHUNKS - NFULL * NW

    def chunk_base(t):
        return (wid + t * NW) * CHUNK

    def idx_desc(t):
        return pltpu.make_async_copy(
            z_hbm.at[pl.ds(chunk_base(t), CHUNK)], idx_v.at[t], isem)

    @pl.when(lax.axis_index("s") == 0)
    def _():
        pltpu.sync_copy(w_hbm, w_sh)

    for t in range(NFULL):
        idx_desc(t).start()

    @pl.when(has_extra)
    def _():
        idx_desc(NFULL).start()

    for t in range(NFULL):
        idx_desc(t).wait()

    @pl.when(has_extra)
    def _():
        idx_desc(NFULL).wait()

    plsc.subcore_barrier()

    def gather_desc(t, u):
        return pltpu.make_async_copy(w_sh.at[idx_v.at[t]], rows[u], gsem[u])

    def out_desc(t, u):
        return pltpu.make_async_copy(
            rows[u], out_hbm.at[pl.ds(chunk_base(t), CHUNK)], osem[u])

    for u in range(NSET * GSZ):
        gather_desc(u, u).start()

    def dgroup(gg, carry):
        t0 = gg * NSET * GSZ
        for s in range(NSET):
            for b in range(GSZ):
                u = s * GSZ + b
                gather_desc(t0 + u, u).wait()
                out_desc(t0 + u, u).start()
        for s in range(NSET):
            for b in range(GSZ):
                u = s * GSZ + b
                t = t0 + NSET * GSZ + u
                @pl.when(t < NFULL)
                def _():
                    out_desc(t - NSET * GSZ, u).wait()
                    gather_desc(t, u).start()
        return carry

    lax.fori_loop(0, NDG, dgroup, 0)

    t0 = NDG * NSET * GSZ
    for b in range(GSZ):
        gather_desc(t0 + b, b).wait()
        out_desc(t0 + b, b).start()
        out_desc(t0 + b, b).wait()
    for b in range(GSZ):
        u = GSZ + b
        out_desc(t0 - GSZ + b, u).wait()

    @pl.when(has_extra)
    def _():
        gather_desc(NFULL, GSZ).start()
        gather_desc(NFULL, GSZ).wait()
        out_desc(NFULL, GSZ).start()
        out_desc(NFULL, GSZ).wait()


def kernel(Z, W):
    w_pad = jnp.concatenate([jnp.zeros((1, EMB), W.dtype), W], axis=0)
    mesh = plsc.VectorSubcoreMesh(core_axis_name="c", subcore_axis_name="s")
    k = functools.partial(
        pl.kernel,
        mesh=mesh,
        out_type=jax.ShapeDtypeStruct((N, EMB), jnp.float32),
        scratch_types=(
            [pltpu.VMEM_SHARED((NROWS, EMB), jnp.float32),
             pltpu.VMEM((NFULL + 1, CHUNK), jnp.int32)]
            + [pltpu.VMEM((CHUNK, EMB), jnp.float32)
               for _ in range(NSET * GSZ)]
            + [pltpu.SemaphoreType.DMA for _ in range(2 * NSET * GSZ + 1)]
        ),
    )(_body)
    return k(w_pad, Z)


# speedup vs baseline: 11.7551x; 1.0184x over previous
import functools

import jax
import jax.numpy as jnp
from jax import lax
from jax.experimental import pallas as pl
from jax.experimental.pallas import tpu as pltpu
from jax.experimental.pallas import tpu_sc as plsc

EMB = 128
NROWS = 65
N = 100000
CHUNK = 80
NCHUNKS = N // CHUNK
NW = 32
GSZ = 3
NSET = 2
NFULL = 39
NG = NFULL // GSZ
NDG = NG // 2


def _body(w_hbm, z_hbm, out_hbm, w_sh, idx_v, *scratch):
    rows = scratch[0:NSET * GSZ]
    gsem = scratch[NSET * GSZ:2 * NSET * GSZ]
    osem = scratch[2 * NSET * GSZ:3 * NSET * GSZ]
    isem = scratch[3 * NSET * GSZ]
    wid = lax.axis_index("s") * 2 + lax.axis_index("c")
    has_extra = wid < NCHUNKS - NFULL * NW

    def chunk_base(t):
        return (wid + t * NW) * CHUNK

    def idx_desc(t):
        return pltpu.make_async_copy(
            z_hbm.at[pl.ds(chunk_base(t), CHUNK)], idx_v.at[t], isem)

    @pl.when(lax.axis_index("s") == 0)
    def _():
        pltpu.sync_copy(w_hbm, w_sh.at[pl.ds(1, NROWS - 1)])

    for t in range(NFULL):
        idx_desc(t).start()

    @pl.when(has_extra)
    def _():
        idx_desc(NFULL).start()

    for t in range(NFULL):
        idx_desc(t).wait()

    @pl.when(has_extra)
    def _():
        idx_desc(NFULL).wait()

    plsc.subcore_barrier()

    def gather_desc(t, u):
        return pltpu.make_async_copy(w_sh.at[idx_v.at[t]], rows[u], gsem[u])

    def out_desc(t, u):
        return pltpu.make_async_copy(
            rows[u], out_hbm.at[pl.ds(chunk_base(t), CHUNK)], osem[u])

    for u in range(NSET * GSZ):
        gather_desc(u, u).start()

    def dgroup(gg, carry):
        t0 = gg * NSET * GSZ
        for s in range(NSET):
            for b in range(GSZ):
                u = s * GSZ + b
                gather_desc(t0 + u, u).wait()
                out_desc(t0 + u, u).start()
        for s in range(NSET):
            for b in range(GSZ):
                u = s * GSZ + b
                t = t0 + NSET * GSZ + u
                @pl.when(t < NFULL)
                def _():
                    out_desc(t - NSET * GSZ, u).wait()
                    gather_desc(t, u).start()
        return carry

    lax.fori_loop(0, NDG, dgroup, 0)

    t0 = NDG * NSET * GSZ
    for b in range(GSZ):
        gather_desc(t0 + b, b).wait()
        out_desc(t0 + b, b).start()
        out_desc(t0 + b, b).wait()
    for b in range(GSZ):
        u = GSZ + b
        out_desc(t0 - GSZ + b, u).wait()

    @pl.when(has_extra)
    def _():
        gather_desc(NFULL, GSZ).start()
        gather_desc(NFULL, GSZ).wait()
        out_desc(NFULL, GSZ).start()
        out_desc(NFULL, GSZ).wait()


def kernel(Z, W):
    mesh = plsc.VectorSubcoreMesh(core_axis_name="c", subcore_axis_name="s")
    k = functools.partial(
        pl.kernel,
        mesh=mesh,
        out_type=jax.ShapeDtypeStruct((N, EMB), jnp.float32),
        scratch_types=(
            [pltpu.VMEM_SHARED((NROWS, EMB), jnp.float32),
             pltpu.VMEM((NFULL + 1, CHUNK), jnp.int32)]
            + [pltpu.VMEM((CHUNK, EMB), jnp.float32)
               for _ in range(NSET * GSZ)]
            + [pltpu.SemaphoreType.DMA for _ in range(2 * NSET * GSZ + 1)]
        ),
    )(_body)
    return k(W, Z)
